# Initial kernel scaffold; baseline (speedup 1.0000x reference)
import jax
import jax.numpy as jnp
from jax.experimental import pallas as pl


def kernel(x, coords, W1, W2):
    raise NotImplementedError("write your pallas kernel here")


# speedup vs baseline: 163.7479x; 163.7479x over previous
import functools

import jax
import jax.numpy as jnp
from jax import lax
from jax.experimental import pallas as pl
from jax.experimental.pallas import tpu as pltpu
from jax.experimental.pallas import tpu_sc as plsc

B = 8
N = 2048
K = 16
CIN = 64
COUT = 128
ROWS = 256


def _h_body(x_ref, w1_ref, w2_ref, h_ref):
    w = jnp.dot(w2_ref[...], w1_ref[...], preferred_element_type=jnp.float32)
    h_ref[0] = jnp.dot(w, x_ref[0], preferred_element_type=jnp.float32)


def _compute_h(x, W1, W2):
    return pl.pallas_call(
        _h_body,
        grid=(B,),
        in_specs=[
            pl.BlockSpec((1, CIN, N), lambda b: (b, 0, 0)),
            pl.BlockSpec((COUT, CIN), lambda b: (0, 0)),
            pl.BlockSpec((COUT, COUT), lambda b: (0, 0)),
        ],
        out_specs=pl.BlockSpec((1, COUT, N), lambda b: (b, 0, 0)),
        out_shape=jax.ShapeDtypeStruct((B, COUT, N), jnp.float32),
    )(x, W1, W2)


def _topk_body(cpad_ref, ct_ref, idx_ref):
    c_all = cpad_ref[0]
    cn = ct_ref[0]
    sq_p = jnp.sum(c_all * c_all, axis=0)
    sq_c = jnp.sum(cn * cn, axis=1, keepdims=True)
    dots = jnp.dot(cn, c_all, preferred_element_type=jnp.float32)
    dist = sq_c + sq_p[None, :] - 2.0 * dots
    iota_m = lax.broadcasted_iota(jnp.int32, (ROWS, N), 1)
    cols = []
    for _ in range(K):
        m = jnp.min(dist, axis=1, keepdims=True)
        idxv = jnp.min(jnp.where(dist == m, iota_m, N), axis=1, keepdims=True)
        cols.append(idxv)
        dist = jnp.where(iota_m == idxv, jnp.float32(jnp.inf), dist)
    idx_ref[0] = jnp.concatenate(cols, axis=1)


def _compute_topk(coords):
    cpad = jnp.pad(coords, ((0, 0), (0, 5), (0, 0)))
    ct = jnp.swapaxes(cpad, 1, 2)
    nrb = N // ROWS
    return pl.pallas_call(
        _topk_body,
        grid=(B, nrb),
        in_specs=[
            pl.BlockSpec((1, 8, N), lambda b, r: (b, 0, 0)),
            pl.BlockSpec((1, ROWS, 8), lambda b, r: (b, r, 0)),
        ],
        out_specs=pl.BlockSpec((1, ROWS, K), lambda b, r: (b, r, 0)),
        out_shape=jax.ShapeDtypeStruct((B, N, K), jnp.int32),
    )(cpad, ct)


def _make_sc_gather():
    mesh = plsc.VectorSubcoreMesh(core_axis_name="c", subcore_axis_name="s")
    info = plsc.get_sparse_core_info()
    nc = info.num_cores
    ns = info.num_subcores
    nw = nc * ns
    wpb = nw // B
    cpw = COUT // wpb

    @functools.partial(
        pl.kernel,
        mesh=mesh,
        compiler_params=pltpu.CompilerParams(needs_layout_passes=False),
        out_type=[
            jax.ShapeDtypeStruct((B, COUT, N), jnp.float32),
            jax.ShapeDtypeStruct((B, COUT, N * K), jnp.float32),
            jax.ShapeDtypeStruct((B, 3, N * K), jnp.float32),
        ],
        scratch_types=[
            pltpu.VMEM((N * K,), jnp.int32),
            pltpu.VMEM((N,), jnp.float32),
            pltpu.VMEM((N * K,), jnp.float32),
            pltpu.VMEM((N,), jnp.float32),
        ],
    )
    def sc_gather(h_hbm, coords_hbm, idx_hbm, y_hbm, knn_hbm, local_hbm,
                  idx_v, row_v, out_v, y_v):
        wid = lax.axis_index("s") * nc + lax.axis_index("c")
        b = wid // wpb
        sub = wid % wpb
        pltpu.sync_copy(idx_hbm.at[b], idx_v)
        lane16 = lax.iota(jnp.int32, 16)
        stride16 = lane16 * 16

        def gather_channel(_unused):
            def p1(i, _):
                idxv = idx_v[pl.ds(i * 16, 16)]
                out_v[pl.ds(i * 16, 16)] = plsc.load_gather(row_v, [idxv])
                return 0
            lax.fori_loop(0, N * K // 16, p1, 0)

        def maxpool(_unused):
            def p2(g, _):
                base = g * 256
                acc = plsc.load_gather(out_v, [stride16 + base])
                for j in range(1, K):
                    acc = jnp.maximum(
                        acc, plsc.load_gather(out_v, [stride16 + (base + j)]))
                y_v[pl.ds(g * 16, 16)] = acc
                return 0
            lax.fori_loop(0, N // 16, p2, 0)

        def h_task(t, _):
            ch = sub * cpw + t
            pltpu.sync_copy(h_hbm.at[b, ch], row_v)
            gather_channel(None)
            maxpool(None)
            pltpu.sync_copy(out_v, knn_hbm.at[b, ch])
            pltpu.sync_copy(y_v, y_hbm.at[b, ch])
            return 0

        lax.fori_loop(0, cpw, h_task, 0)

        @pl.when(sub < 3)
        def coord_task():
            d = sub
            pltpu.sync_copy(coords_hbm.at[b, d], row_v)

            def p1(g, _):
                centers = row_v[pl.ds(g * 16, 16)]
                for t in range(16):
                    i = g * 16 + t
                    idxv = idx_v[pl.ds(i * 16, 16)]
                    vals = plsc.load_gather(row_v, [idxv])
                    out_v[pl.ds(i * 16, 16)] = centers[t] - vals
                return 0
            lax.fori_loop(0, N // 16, p1, 0)
            pltpu.sync_copy(out_v, local_hbm.at[b, d])

    return sc_gather


_SC_CACHE = []


def kernel(x, coords, W1, W2):
    h = _compute_h(x, W1, W2)
    idx = _compute_topk(coords)
    if not _SC_CACHE:
        _SC_CACHE.append(_make_sc_gather())
    y, knn_flat, local_flat = _SC_CACHE[0](h, coords, idx.reshape(B, N * K))
    return (y,
            knn_flat.reshape(B, COUT, N, K),
            local_flat.reshape(B, 3, N, K))
